# baseline (speedup 1.0000x reference)
import functools

import jax
import jax.numpy as jnp
from jax.experimental import pallas as pl
from jax.experimental.pallas import tpu as pltpu

THRESHOLD = 0.05
WEIGHT = 1.0

B = 8
N = 16384
M = 2048
BN = 2048
NBLK = N // BN


def _loss_kernel(c_ref, pt_ref, out_ref, acc_ref):
    b = pl.program_id(0)
    i = pl.program_id(1)
    step = b * NBLK + i

    @pl.when(step == 0)
    def _init():
        acc_ref[0] = 0.0
        acc_ref[1] = 0.0

    c = c_ref[0]
    pt = pt_ref[0]

    cross = jnp.dot(c, pt, preferred_element_type=jnp.float32)
    c2 = jnp.sum(c * c, axis=1, keepdims=True)
    p2 = jnp.sum(pt * pt, axis=0, keepdims=True)
    d = jnp.maximum(c2 + p2 - 2.0 * cross, 0.0)
    dmin = jnp.min(d, axis=1)

    mask = dmin < THRESHOLD
    acc_ref[0] += jnp.sum(jnp.where(mask, dmin, 0.0))
    acc_ref[1] += jnp.sum(mask.astype(jnp.float32))

    @pl.when(step == B * NBLK - 1)
    def _finish():
        s = acc_ref[0]
        m = acc_ref[1]
        out_ref[0, 0] = jnp.where(m > 0.0, s / (m + 1e-6), 0.0)


@jax.jit
def kernel(completed, partial):
    cpad = jnp.pad(completed, ((0, 0), (0, 0), (0, 5)))
    ppad = jnp.pad(partial, ((0, 0), (0, 0), (0, 5)))
    pt = jnp.transpose(ppad, (0, 2, 1))

    out = pl.pallas_call(
        _loss_kernel,
        grid=(B, NBLK),
        in_specs=[
            pl.BlockSpec((1, BN, 8), lambda b, i: (b, i, 0)),
            pl.BlockSpec((1, 8, M), lambda b, i: (b, 0, 0)),
        ],
        out_specs=pl.BlockSpec(memory_space=pltpu.SMEM),
        out_shape=jax.ShapeDtypeStruct((1, 1), jnp.float32),
        scratch_shapes=[pltpu.SMEM((2,), jnp.float32)],
    )(cpad, pt)
    return WEIGHT * out[0, 0]


# speedup vs baseline: 1.2102x; 1.2102x over previous
import jax
import jax.numpy as jnp
from jax.experimental import pallas as pl
from jax.experimental.pallas import tpu as pltpu

THRESHOLD = 0.05
WEIGHT = 1.0

B = 8
N = 16384
M = 2048
BN = 2048
NBLK = N // BN


def _loss_kernel(c_ref, pt_ref, out_ref, acc_ref):
    b = pl.program_id(0)
    i = pl.program_id(1)
    step = b * NBLK + i

    @pl.when(step == 0)
    def _init():
        acc_ref[0] = 0.0
        acc_ref[1] = 0.0

    c = c_ref[0]
    pt = pt_ref[0]

    d = jnp.dot(c, pt, preferred_element_type=jnp.float32)
    dmin = jnp.min(d, axis=1)

    mask = dmin < THRESHOLD
    acc_ref[0] += jnp.sum(jnp.where(mask, dmin, 0.0))
    acc_ref[1] += jnp.sum(mask.astype(jnp.float32))

    @pl.when(step == B * NBLK - 1)
    def _finish():
        s = acc_ref[0]
        m = acc_ref[1]
        out_ref[0, 0] = jnp.where(m > 0.0, s / (m + 1e-6), 0.0)


@jax.jit
def kernel(completed, partial):
    c2 = jnp.sum(completed * completed, axis=-1, keepdims=True)
    ones_c = jnp.ones_like(c2)
    caug = jnp.concatenate([completed, c2, ones_c], axis=-1)
    caug = jnp.pad(caug, ((0, 0), (0, 0), (0, 3)))

    p2 = jnp.sum(partial * partial, axis=-1, keepdims=True)
    ones_p = jnp.ones_like(p2)
    paug = jnp.concatenate([-2.0 * partial, ones_p, p2], axis=-1)
    paug = jnp.pad(paug, ((0, 0), (0, 0), (0, 3)))
    pt = jnp.transpose(paug, (0, 2, 1))

    out = pl.pallas_call(
        _loss_kernel,
        grid=(B, NBLK),
        in_specs=[
            pl.BlockSpec((1, BN, 8), lambda b, i: (b, i, 0)),
            pl.BlockSpec((1, 8, M), lambda b, i: (b, 0, 0)),
        ],
        out_specs=pl.BlockSpec(memory_space=pltpu.SMEM),
        out_shape=jax.ShapeDtypeStruct((1, 1), jnp.float32),
        scratch_shapes=[pltpu.SMEM((2,), jnp.float32)],
    )(caug, pt)
    return WEIGHT * out[0, 0]
